# baseline (speedup 1.0000x reference)
import jax
import jax.numpy as jnp
from jax import lax
from jax.experimental import pallas as pl
from jax.experimental.pallas import tpu as pltpu
from jax.experimental.pallas import tpu_sc as plsc

BATCH, SEQ, D = 4, 4096, 2048
NC, NS = 2, 16
NW = NC * NS
SEQ_PER_W = SEQ // NW
CS = 2
NCHUNK = SEQ_PER_W // CS
NBUF = 4
LANES = 16
VPR = D // LANES


def _sc_body(x_hbm, emb_hbm, out_hbm, xbuf, ebuf, insem, outsem):
    wid = lax.axis_index("s") * NC + lax.axis_index("c")
    s_base = wid * SEQ_PER_W

    def in_copies(ci, k):
        s0 = s_base + ci * CS
        return (
            pltpu.make_async_copy(
                x_hbm.at[:, pl.ds(s0, CS)], xbuf.at[k], insem.at[k]
            ),
            pltpu.make_async_copy(
                emb_hbm.at[pl.ds(s0, CS)], ebuf.at[k], insem.at[k]
            ),
        )

    def out_copy(ci, k):
        s0 = s_base + ci * CS
        return pltpu.make_async_copy(
            xbuf.at[k], out_hbm.at[:, pl.ds(s0, CS)], outsem.at[k]
        )

    def start_in(ci, k):
        for c in in_copies(ci, k):
            c.start()

    def wait_in(ci, k):
        for c in in_copies(ci, k):
            c.wait()

    def compute(k):
        @plsc.parallel_loop(0, VPR, step=1, unroll=4)
        def vec(j):
            off = j * LANES
            for s in range(CS):
                e = ebuf[k, s, pl.ds(off, LANES)]
                for b in range(BATCH):
                    xbuf[k, b, s, pl.ds(off, LANES)] = (
                        xbuf[k, b, s, pl.ds(off, LANES)] + e
                    )

    start_in(0, 0)

    def step(ci, carry):
        k = lax.rem(ci, NBUF)
        kn = lax.rem(ci + 1, NBUF)

        @pl.when(jnp.logical_and(ci + 1 < NCHUNK, ci >= NBUF - 1))
        def _():
            out_copy(ci + 1 - NBUF, kn).wait()

        @pl.when(ci + 1 < NCHUNK)
        def _():
            start_in(ci + 1, kn)

        wait_in(ci, k)
        compute(k)
        out_copy(ci, k).start()
        return carry

    lax.fori_loop(0, NCHUNK, step, 0)
    for ci in range(NCHUNK - NBUF, NCHUNK):
        out_copy(ci, ci % NBUF).wait()


INTERPRET = False


def kernel(x, embedding):
    mesh = plsc.VectorSubcoreMesh(
        core_axis_name="c", subcore_axis_name="s", num_cores=NC, num_subcores=NS
    )
    f = pl.kernel(
        _sc_body,
        jax.ShapeDtypeStruct((BATCH, SEQ, D), jnp.float32),
        mesh=mesh,
        scratch_types=[
            pltpu.VMEM((NBUF, BATCH, CS, D), jnp.float32),
            pltpu.VMEM((NBUF, CS, D), jnp.float32),
            pltpu.SemaphoreType.DMA((NBUF,)),
            pltpu.SemaphoreType.DMA((NBUF,)),
        ],
        interpret=INTERPRET,
    )
    return f(x, embedding)


# speedup vs baseline: 1.0000x; 1.0000x over previous
import jax
import jax.numpy as jnp
from jax import lax
from jax.experimental import pallas as pl
from jax.experimental.pallas import tpu as pltpu
from jax.experimental.pallas import tpu_sc as plsc

BATCH, SEQ, D = 4, 4096, 2048
NC, NS = 2, 16
NW = NC * NS
SEQ_PER_W = SEQ // NW
CS = 4
NCHUNK = SEQ_PER_W // CS
NBUF = 3
LANES = 16
VPR = D // LANES


def _sc_body(x_hbm, emb_hbm, out_hbm, xbuf, ebuf, insem, outsem):
    wid = lax.axis_index("s") * NC + lax.axis_index("c")
    s_base = wid * SEQ_PER_W

    def in_copies(ci, k):
        s0 = s_base + ci * CS
        copies = [
            pltpu.make_async_copy(
                x_hbm.at[b, pl.ds(s0, CS)], xbuf.at[k, b], insem.at[k]
            )
            for b in range(BATCH)
        ]
        copies.append(
            pltpu.make_async_copy(
                emb_hbm.at[pl.ds(s0, CS)], ebuf.at[k], insem.at[k]
            )
        )
        return copies

    def out_copies(ci, k):
        s0 = s_base + ci * CS
        return [
            pltpu.make_async_copy(
                xbuf.at[k, b], out_hbm.at[b, pl.ds(s0, CS)], outsem.at[k]
            )
            for b in range(BATCH)
        ]

    def start_in(ci, k):
        for c in in_copies(ci, k):
            c.start()

    def wait_in(ci, k):
        for c in in_copies(ci, k):
            c.wait()

    def compute(k):
        @plsc.parallel_loop(0, VPR, step=1, unroll=4)
        def vec(j):
            off = j * LANES
            for s in range(CS):
                e = ebuf[k, s, pl.ds(off, LANES)]
                for b in range(BATCH):
                    xbuf[k, b, s, pl.ds(off, LANES)] = (
                        xbuf[k, b, s, pl.ds(off, LANES)] + e
                    )

    start_in(0, 0)

    def step(ci, carry):
        k = lax.rem(ci, NBUF)
        kn = lax.rem(ci + 1, NBUF)

        @pl.when(jnp.logical_and(ci + 1 < NCHUNK, ci >= NBUF - 1))
        def _():
            for c in out_copies(ci + 1 - NBUF, kn):
                c.wait()

        @pl.when(ci + 1 < NCHUNK)
        def _():
            start_in(ci + 1, kn)

        wait_in(ci, k)
        compute(k)
        for c in out_copies(ci, k):
            c.start()
        return carry

    lax.fori_loop(0, NCHUNK, step, 0)
    for ci in range(NCHUNK - NBUF, NCHUNK):
        for c in out_copies(ci, ci % NBUF):
            c.wait()


INTERPRET = False


def kernel(x, embedding):
    mesh = plsc.VectorSubcoreMesh(
        core_axis_name="c", subcore_axis_name="s", num_cores=NC, num_subcores=NS
    )
    f = pl.kernel(
        _sc_body,
        jax.ShapeDtypeStruct((BATCH, SEQ, D), jnp.float32),
        mesh=mesh,
        scratch_types=[
            pltpu.VMEM((NBUF, BATCH, CS, D), jnp.float32),
            pltpu.VMEM((NBUF, CS, D), jnp.float32),
            pltpu.SemaphoreType.DMA((NBUF,)),
            pltpu.SemaphoreType.DMA((NBUF,)),
        ],
        interpret=INTERPRET,
    )
    return f(x, embedding)
